# baseline (speedup 1.0000x reference)
import functools

import jax
import jax.numpy as jnp
from jax import lax
from jax.experimental import pallas as pl
from jax.experimental.pallas import tpu as pltpu
from jax.experimental.pallas import tpu_sc as plsc

B = 4096
D = 64
LANES = 16
NC, NS = 2, 16
NW = NC * NS
BPW = B // NW
LP = 208
GP = 64
LC = LP // 2
DC = D // LANES


def _sc_pool_body(word_table, graph_table, widx_hbm, gidx_hbm, out_hbm,
                  widx_v, gidx_v, wbuf, gbuf, wrow0, grow0, out_v, sem):
    wid = lax.axis_index("s") * NC + lax.axis_index("c")
    base = wid * BPW

    pltpu.sync_copy(widx_hbm.at[pl.ds(base, BPW)], widx_v)
    pltpu.sync_copy(gidx_hbm.at[pl.ds(base, BPW)], gidx_v)
    pltpu.sync_copy(word_table.at[pl.ds(0, 1)], wrow0)
    pltpu.sync_copy(graph_table.at[pl.ds(0, 1)], grow0)

    def per_sample(i, carry):
        c0 = pltpu.async_copy(word_table.at[widx_v.at[i, pl.ds(0, LC)]],
                              wbuf.at[pl.ds(0, LC)], sem)
        c1 = pltpu.async_copy(word_table.at[widx_v.at[i, pl.ds(LC, LC)]],
                              wbuf.at[pl.ds(LC, LC)], sem)
        c2 = pltpu.async_copy(graph_table.at[gidx_v.at[i]], gbuf, sem)
        c0.wait()
        c1.wait()
        c2.wait()

        def count_zeros(idx_v, nchunks):
            def cbody(k, acc):
                chunk = idx_v[i, pl.ds(k * LANES, LANES)]
                return acc + jnp.where(chunk == 0, 1, 0).astype(jnp.int32)
            acc = lax.fori_loop(0, nchunks,
                                cbody, jnp.zeros((LANES,), jnp.int32))
            return jnp.sum(acc)

        n0w = count_zeros(widx_v, LP // LANES)
        n0g = count_zeros(gidx_v, GP // LANES)

        def wsum(r, accs):
            return tuple(accs[c] + wbuf[r, pl.ds(c * LANES, LANES)]
                         for c in range(DC))

        def gsum(r, accs):
            return tuple(accs[c] + gbuf[r, pl.ds(c * LANES, LANES)]
                         for c in range(DC))

        zeros = tuple(jnp.zeros((LANES,), jnp.float32) for _ in range(DC))
        waccs = lax.fori_loop(0, LP, wsum, zeros)
        gaccs = lax.fori_loop(0, GP, gsum, zeros)

        ones = jnp.ones((LANES,), jnp.float32)
        n0w_f = jnp.full((LANES,), n0w, jnp.int32).astype(jnp.float32)
        n0g_f = jnp.full((LANES,), n0g, jnp.int32).astype(jnp.float32)
        inv_w = ones / jnp.maximum(jnp.float32(LP) - n0w_f, ones)
        inv_g = ones / jnp.maximum(jnp.float32(GP) - n0g_f, ones)
        for c in range(DC):
            sl = pl.ds(c * LANES, LANES)
            mw = (waccs[c] - n0w_f * wrow0[0, sl]) * inv_w
            mg = (gaccs[c] - n0g_f * grow0[0, sl]) * inv_g
            out_v[i, sl] = mw + mg
        return carry

    lax.fori_loop(0, BPW, per_sample, 0)
    pltpu.sync_copy(out_v, out_hbm.at[pl.ds(base, BPW)])


def _sc_pool(widx, gidx, word_table, graph_table):
    mesh = plsc.VectorSubcoreMesh(core_axis_name="c", subcore_axis_name="s",
                                  num_cores=NC, num_subcores=NS)
    kern = pl.kernel(
        _sc_pool_body,
        out_type=jax.ShapeDtypeStruct((B, D), jnp.float32),
        mesh=mesh,
        scratch_types=[
            pltpu.VMEM((BPW, LP), jnp.int32),
            pltpu.VMEM((BPW, GP), jnp.int32),
            pltpu.VMEM((LP, D), jnp.float32),
            pltpu.VMEM((GP, D), jnp.float32),
            pltpu.VMEM((1, D), jnp.float32),
            pltpu.VMEM((1, D), jnp.float32),
            pltpu.VMEM((BPW, D), jnp.float32),
            pltpu.SemaphoreType.DMA,
        ],
        compiler_params=pltpu.CompilerParams(use_tc_tiling_on_sc=False,
                                             needs_layout_passes=False),
    )
    return kern(word_table, graph_table, widx, gidx)


def _mlp_body(x_ref, w1_ref, b1_ref, w2_ref, b2_ref, w3_ref, b3_ref, o_ref):
    x = x_ref[...]
    h = jnp.maximum(
        jnp.dot(x, w1_ref[...], preferred_element_type=jnp.float32)
        + b1_ref[...], 0.0)
    h = jnp.maximum(
        jnp.dot(h, w2_ref[...], preferred_element_type=jnp.float32)
        + b2_ref[...], 0.0)
    o_ref[...] = (jnp.dot(h, w3_ref[...], preferred_element_type=jnp.float32)
                  + b3_ref[...])


def _mlp(x, W1, b1, W2, b2, W3, b3):
    H = W1.shape[1]
    O = W3.shape[1]
    blk = 512
    grid = (B // blk,)
    return pl.pallas_call(
        _mlp_body,
        grid=grid,
        in_specs=[
            pl.BlockSpec((blk, D), lambda i: (i, 0)),
            pl.BlockSpec((D, H), lambda i: (0, 0)),
            pl.BlockSpec((1, H), lambda i: (0, 0)),
            pl.BlockSpec((H, H), lambda i: (0, 0)),
            pl.BlockSpec((1, H), lambda i: (0, 0)),
            pl.BlockSpec((H, O), lambda i: (0, 0)),
            pl.BlockSpec((1, O), lambda i: (0, 0)),
        ],
        out_specs=pl.BlockSpec((blk, O), lambda i: (i, 0)),
        out_shape=jax.ShapeDtypeStruct((B, O), jnp.float32),
    )(x, W1, b1.reshape(1, H), W2, b2.reshape(1, H), W3, b3.reshape(1, O))


def kernel(input, graph, word_table, graph_table, alpha, beta,
           W1, b1, W2, b2, W3, b3):
    widx = jnp.pad(input, ((0, 0), (0, LP - input.shape[1])))
    gidx = jnp.pad(graph, ((0, 0), (0, GP - graph.shape[1])))
    combined = _sc_pool(widx, gidx, word_table, graph_table)
    return _mlp(combined, W1, b1, W2, b2, W3, b3)
